# baseline (speedup 1.0000x reference)
import jax
import jax.numpy as jnp
from jax.experimental import pallas as pl

L, D, H, DH = 2048, 768, 12, 64
LOCAL, FUT, THR = 64, 32, 0.5
NEG = -1e9
BQ = 256
NI = L // BQ


def _stats_kernel(x_ref, xhat_ref, lw_ref, fw_ref, kp_ref, cm_ref):
    x = x_ref[...]
    m = jnp.mean(x, axis=1, keepdims=True)
    v = jnp.mean((x - m) ** 2, axis=1, keepdims=True)
    xhat_ref[...] = (x - m) / jnp.sqrt(v + 1e-5)

    pos = jax.lax.broadcasted_iota(jnp.int32, (L, 1), 0).astype(jnp.float32)

    def sdiff(s):
        d = jnp.abs(jnp.roll(x, -s, axis=0) - x)
        return jnp.where(pos < (L - s), d, 0.0)

    d1 = sdiff(1)
    d2 = sdiff(2)
    d3 = sdiff(3)
    d4 = sdiff(4)
    d5 = sdiff(5)

    comb = d1 * 0.5 + (d2 / 2) * 0.3 + (d4 / 4) * 0.2
    imp = jnp.mean(comb, axis=1, keepdims=True)
    imp = (imp - jnp.min(imp)) / (jnp.max(imp) - jnp.min(imp) + 1e-6)
    lw_ref[...] = jnp.clip(jnp.round(LOCAL * (0.5 + 0.5 * imp)), 2, 2 * LOCAL)

    xbar = jnp.mean(x, axis=0, keepdims=True)
    t = jnp.abs(jnp.roll(x, -1, axis=0) - xbar)
    t = jnp.where(pos < (L - 1), t, 0.0)
    ti = jnp.mean(t, axis=1, keepdims=True)
    ti = (ti - jnp.min(ti)) / (jnp.max(ti) - jnp.min(ti) + 1e-6)
    fw_ref[...] = jnp.clip(jnp.round(FUT * (0.5 + 0.5 * ti)), 1, FUT)

    cd = jnp.mean(d1 * 0.4 + (d2 / 2) * 0.3 + (d3 / 3) * 0.2 + (d5 / 5) * 0.1,
                  axis=1, keepdims=True)
    cmean = jnp.mean(cd)
    cvar = jnp.sum((cd - cmean) ** 2) / (L - 1)
    thr = cmean + THR * jnp.sqrt(cvar)
    interior = (cd > jnp.roll(cd, 1, axis=0)) & (cd > jnp.roll(cd, -1, axis=0))
    edge = (pos == 0) | (pos == (L - 1))
    kp = (cd > thr) & (interior | edge)
    kp_f = jnp.where(kp, 1.0, 0.0)
    kp_ref[...] = kp_f
    has_f = jnp.max(kp_f)

    segw = L // 4
    segid = jnp.floor(pos / segw)
    sel_f = jnp.zeros((L, 1), jnp.float32)
    for g in range(4):
        for _ in range(4):
            avail = (segid == g) & (sel_f < 0.5)
            vals = jnp.where(avail, imp, -jnp.inf)
            mx = jnp.max(vals)
            idx = jnp.min(jnp.where(vals == mx, pos, float(L)))
            sel_f = jnp.maximum(sel_f, jnp.where(pos == idx, 1.0, 0.0))

    fb_f = jnp.where(
        (pos == 0) | (pos == 511) | (pos == 1023)
        | (pos == 1535) | (pos == 2047), 1.0, 0.0)
    cm_ref[...] = jnp.maximum(jnp.maximum(kp_f, sel_f),
                              fb_f * (1.0 - has_f))


def _qkv_kernel(xh_ref, w_ref, b_ref, out_ref):
    out_ref[...] = (
        jnp.dot(xh_ref[...], w_ref[...], preferred_element_type=jnp.float32)
        + b_ref[...]
    )


def _attn_kernel(lw_ref, fw_ref, kp_ref, cm_ref, q_ref, k_ref, v_ref, o_ref):
    i = pl.program_id(1)
    q = q_ref[0]
    k = k_ref[0]
    s = jax.lax.dot_general(
        q, k, (((1,), (1,)), ((), ())), preferred_element_type=jnp.float32
    ) * 0.125
    rows = (i * BQ
            + jax.lax.broadcasted_iota(jnp.int32, (BQ, 1), 0)).astype(jnp.float32)
    cols = jax.lax.broadcasted_iota(jnp.int32, (1, L), 1).astype(jnp.float32)
    win = (cols >= rows - lw_ref[...]) & (cols <= rows + fw_ref[...])
    mask = win | (kp_ref[...] > 0) | (cm_ref[...] > 0) | (cols == rows)
    s = jnp.where(mask, s, NEG)
    mx = jnp.max(s, axis=1, keepdims=True)
    p = jnp.exp(s - mx)
    denom = jnp.sum(p, axis=1, keepdims=True)
    o_ref[0] = (
        jnp.dot(p, v_ref[0], preferred_element_type=jnp.float32) / denom
    )


def _out_kernel(o_ref, x_ref, wo_ref, bo_ref, wg_ref, bg_ref, y_ref):
    gate = jax.nn.sigmoid(
        jnp.dot(x_ref[...], wg_ref[...], preferred_element_type=jnp.float32)
        + bg_ref[...]
    )
    y = (
        jnp.dot(o_ref[...], wo_ref[...], preferred_element_type=jnp.float32)
        + bo_ref[...]
    )
    y_ref[...] = gate * y


def kernel(x, Wq, bq, Wk, bk, Wv, bv, Wo, bo,
           gnq, bnq, gnk, bnk, gnv, bnv, Wg, bg):
    x2 = x.reshape(L, D)
    Wq_e = gnq[:, None] * Wq.T
    Wk_e = gnk[:, None] * Wk.T
    Wv_e = gnv[:, None] * Wv.T
    Wqkv = jnp.concatenate([Wq_e, Wk_e, Wv_e], axis=1)
    bqkv = jnp.concatenate([bq + bnq @ Wq.T, bk + bnk @ Wk.T,
                            bv + bnv @ Wv.T])[None, :]

    xhat, lw, fw, kp, cm = pl.pallas_call(
        _stats_kernel,
        out_shape=(
            jax.ShapeDtypeStruct((L, D), jnp.float32),
            jax.ShapeDtypeStruct((L, 1), jnp.float32),
            jax.ShapeDtypeStruct((L, 1), jnp.float32),
            jax.ShapeDtypeStruct((L, 1), jnp.float32),
            jax.ShapeDtypeStruct((L, 1), jnp.float32),
        ),
    )(x2)

    qkv = pl.pallas_call(
        _qkv_kernel,
        grid=(NI,),
        in_specs=[
            pl.BlockSpec((BQ, D), lambda i: (i, 0)),
            pl.BlockSpec((D, 3 * D), lambda i: (0, 0)),
            pl.BlockSpec((1, 3 * D), lambda i: (0, 0)),
        ],
        out_specs=pl.BlockSpec((BQ, 3 * D), lambda i: (i, 0)),
        out_shape=jax.ShapeDtypeStruct((L, 3 * D), jnp.float32),
    )(xhat, Wqkv, bqkv)

    cmr = cm.reshape(1, L)
    qh = qkv[:, :D].reshape(L, H, DH).transpose(1, 0, 2)
    kh = qkv[:, D:2 * D].reshape(L, H, DH).transpose(1, 0, 2)
    vh = qkv[:, 2 * D:].reshape(L, H, DH).transpose(1, 0, 2)
    oh = pl.pallas_call(
        _attn_kernel,
        grid=(H, NI),
        in_specs=[
            pl.BlockSpec((BQ, 1), lambda h, i: (i, 0)),
            pl.BlockSpec((BQ, 1), lambda h, i: (i, 0)),
            pl.BlockSpec((BQ, 1), lambda h, i: (i, 0)),
            pl.BlockSpec((1, L), lambda h, i: (0, 0)),
            pl.BlockSpec((1, BQ, DH), lambda h, i: (h, i, 0)),
            pl.BlockSpec((1, L, DH), lambda h, i: (h, 0, 0)),
            pl.BlockSpec((1, L, DH), lambda h, i: (h, 0, 0)),
        ],
        out_specs=pl.BlockSpec((1, BQ, DH), lambda h, i: (h, i, 0)),
        out_shape=jax.ShapeDtypeStruct((H, L, DH), jnp.float32),
    )(lw, fw, kp, cmr, qh, kh, vh)
    o = oh.transpose(1, 0, 2).reshape(L, D)

    y = pl.pallas_call(
        _out_kernel,
        grid=(NI,),
        in_specs=[
            pl.BlockSpec((BQ, D), lambda i: (i, 0)),
            pl.BlockSpec((BQ, D), lambda i: (i, 0)),
            pl.BlockSpec((D, D), lambda i: (0, 0)),
            pl.BlockSpec((1, D), lambda i: (0, 0)),
            pl.BlockSpec((D, D), lambda i: (0, 0)),
            pl.BlockSpec((1, D), lambda i: (0, 0)),
        ],
        out_specs=pl.BlockSpec((BQ, D), lambda i: (i, 0)),
        out_shape=jax.ShapeDtypeStruct((L, D), jnp.float32),
    )(o, x2, Wo.T, bo[None], Wg.T, bg[None])

    return y.reshape(1, L, D)


# speedup vs baseline: 2.2166x; 2.2166x over previous
import jax
import jax.numpy as jnp
from jax.experimental import pallas as pl
from jax.experimental.pallas import tpu as pltpu

L, D, H, DH = 2048, 768, 12, 64
LOCAL, FUT, THR = 64, 32, 0.5
NEG = -1e9
BQ = 1024
NI = L // BQ


def _stats_kernel(x_ref, xhat_ref, lw_ref, fw_ref, kp_ref, cm_ref):
    x = x_ref[...]
    m = jnp.mean(x, axis=1, keepdims=True)
    xc = x - m
    v = jnp.mean(xc * xc, axis=1, keepdims=True)
    xhat_ref[...] = xc / jnp.sqrt(v + 1e-5)

    pos = jax.lax.broadcasted_iota(jnp.int32, (L, 1), 0).astype(jnp.float32)
    rolls = {s: jnp.roll(x, -s, axis=0) for s in (1, 2, 3, 4, 5)}

    def sdm(s):
        d = jnp.mean(jnp.abs(rolls[s] - x), axis=1, keepdims=True)
        return jnp.where(pos < (L - s), d, 0.0)

    s1, s2, s3, s4, s5 = sdm(1), sdm(2), sdm(3), sdm(4), sdm(5)
    imp = s1 * 0.5 + s2 * 0.15 + s4 * 0.05
    cd = s1 * 0.4 + s2 * 0.15 + s3 * (0.2 / 3.0) + s5 * 0.02

    xbar = jnp.mean(x, axis=0, keepdims=True)
    ti = jnp.mean(jnp.abs(rolls[1] - xbar), axis=1, keepdims=True)
    ti = jnp.where(pos < (L - 1), ti, 0.0)

    imp = jnp.transpose(imp)
    cd = jnp.transpose(cd)
    ti = jnp.transpose(ti)
    posr = jax.lax.broadcasted_iota(jnp.int32, (1, L), 1).astype(jnp.float32)

    imp = (imp - jnp.min(imp)) / (jnp.max(imp) - jnp.min(imp) + 1e-6)
    lw = jnp.clip(jnp.round(LOCAL * (0.5 + 0.5 * imp)), 2, 2 * LOCAL)
    lw_ref[...] = jnp.transpose(lw)

    ti = (ti - jnp.min(ti)) / (jnp.max(ti) - jnp.min(ti) + 1e-6)
    fw = jnp.clip(jnp.round(FUT * (0.5 + 0.5 * ti)), 1, FUT)
    fw_ref[...] = jnp.transpose(fw)

    cmean = jnp.mean(cd)
    cvar = jnp.sum((cd - cmean) ** 2) / (L - 1)
    thr = cmean + THR * jnp.sqrt(cvar)
    interior = (cd > jnp.roll(cd, 1, axis=1)) & (cd > jnp.roll(cd, -1, axis=1))
    edge = (posr == 0) | (posr == (L - 1))
    kp = (cd > thr) & (interior | edge)
    kp_f = jnp.where(kp, 1.0, 0.0)
    kp_ref[...] = jnp.transpose(kp_f)
    has_f = jnp.max(kp_f)

    segid = jnp.floor(posr * (4.0 / L))
    sel_f = jnp.zeros((1, L), jnp.float32)
    for g in range(4):
        for _ in range(4):
            avail = (segid == g) & (sel_f < 0.5)
            vals = jnp.where(avail, imp, -jnp.inf)
            mx = jnp.max(vals)
            idx = jnp.min(jnp.where(vals == mx, posr, float(L)))
            sel_f = jnp.maximum(sel_f, jnp.where(posr == idx, 1.0, 0.0))

    fb_f = jnp.where(
        (posr == 0) | (posr == 511) | (posr == 1023)
        | (posr == 1535) | (posr == 2047), 1.0, 0.0)
    cm_ref[...] = jnp.maximum(jnp.maximum(kp_f, sel_f),
                              fb_f * (1.0 - has_f))


def _qkv_kernel(xh_ref, w_ref, b_ref, out_ref):
    out_ref[...] = (
        jnp.dot(xh_ref[...], w_ref[...], preferred_element_type=jnp.float32)
        + b_ref[...]
    )


NP = H // 2


def _attn_kernel(lw_ref, fw_ref, kp_ref, cm_ref, q_ref, k_ref, v_ref,
                 x_ref, wo_ref, bo_ref, wg_ref, bg_ref, y_ref,
                 bias_ref, oacc_ref):
    i = pl.program_id(0)
    p_id = pl.program_id(1)

    @pl.when(p_id == 0)
    def _build_bias():
        rows = (i * BQ + jax.lax.broadcasted_iota(jnp.int32, (BQ, 1), 0)
                ).astype(jnp.float32)
        cols = jax.lax.broadcasted_iota(jnp.int32, (1, L), 1).astype(jnp.float32)
        win = (cols >= rows - lw_ref[...]) & (cols <= rows + fw_ref[...])
        mask = win | (kp_ref[...] > 0) | (cm_ref[...] > 0) | (cols == rows)
        bias_ref[...] = jnp.where(mask, 0.0, NEG)

    q2 = q_ref[...]
    k2 = k_ref[...]
    v2 = v_ref[...]
    bias = bias_ref[...]
    outs = []
    for half in range(2):
        q = q2[:, half * DH:(half + 1) * DH]
        k = k2[:, half * DH:(half + 1) * DH]
        v = v2[:, half * DH:(half + 1) * DH]
        s = jax.lax.dot_general(
            q, k, (((1,), (1,)), ((), ())), preferred_element_type=jnp.float32
        ) + bias
        mx = jnp.max(s, axis=1, keepdims=True)
        p = jnp.exp(s - mx)
        denom = jnp.sum(p, axis=1, keepdims=True)
        outs.append(
            jnp.dot(p, v, preferred_element_type=jnp.float32) / denom)
    oacc_ref[p_id] = jnp.concatenate(outs, axis=1)

    @pl.when(p_id == NP - 1)
    def _finalize():
        o = jnp.concatenate([oacc_ref[j] for j in range(NP)], axis=1)
        gate = jax.nn.sigmoid(
            jnp.dot(x_ref[...], wg_ref[...],
                    preferred_element_type=jnp.float32) + bg_ref[...])
        y = jnp.dot(o, wo_ref[...],
                    preferred_element_type=jnp.float32) + bo_ref[...]
        y_ref[...] = gate * y


def kernel(x, Wq, bq, Wk, bk, Wv, bv, Wo, bo,
           gnq, bnq, gnk, bnk, gnv, bnv, Wg, bg):
    x2 = x.reshape(L, D)
    Wq_e = 0.125 * (gnq[:, None] * Wq.T)
    Wk_e = gnk[:, None] * Wk.T
    Wv_e = gnv[:, None] * Wv.T
    Wqkv = jnp.concatenate([Wq_e, Wk_e, Wv_e], axis=1)
    bqkv = jnp.concatenate([0.125 * (bq + bnq @ Wq.T), bk + bnk @ Wk.T,
                            bv + bnv @ Wv.T])[None, :]

    xhat, lw, fw, kp, cmr = pl.pallas_call(
        _stats_kernel,
        out_shape=(
            jax.ShapeDtypeStruct((L, D), jnp.float32),
            jax.ShapeDtypeStruct((L, 1), jnp.float32),
            jax.ShapeDtypeStruct((L, 1), jnp.float32),
            jax.ShapeDtypeStruct((L, 1), jnp.float32),
            jax.ShapeDtypeStruct((1, L), jnp.float32),
        ),
    )(x2)

    qkv = pl.pallas_call(
        _qkv_kernel,
        grid=(NI,),
        in_specs=[
            pl.BlockSpec((BQ, D), lambda i: (i, 0)),
            pl.BlockSpec((D, 3 * D), lambda i: (0, 0)),
            pl.BlockSpec((1, 3 * D), lambda i: (0, 0)),
        ],
        out_specs=pl.BlockSpec((BQ, 3 * D), lambda i: (i, 0)),
        out_shape=jax.ShapeDtypeStruct((L, 3 * D), jnp.float32),
    )(xhat, Wqkv, bqkv)

    y = pl.pallas_call(
        _attn_kernel,
        grid=(NI, NP),
        in_specs=[
            pl.BlockSpec((BQ, 1), lambda i, p: (i, 0)),
            pl.BlockSpec((BQ, 1), lambda i, p: (i, 0)),
            pl.BlockSpec((BQ, 1), lambda i, p: (i, 0)),
            pl.BlockSpec((1, L), lambda i, p: (0, 0)),
            pl.BlockSpec((BQ, 2 * DH), lambda i, p: (i, p)),
            pl.BlockSpec((L, 2 * DH), lambda i, p: (0, NP + p)),
            pl.BlockSpec((L, 2 * DH), lambda i, p: (0, 2 * NP + p)),
            pl.BlockSpec((BQ, D), lambda i, p: (i, 0)),
            pl.BlockSpec((D, D), lambda i, p: (0, 0)),
            pl.BlockSpec((1, D), lambda i, p: (0, 0)),
            pl.BlockSpec((D, D), lambda i, p: (0, 0)),
            pl.BlockSpec((1, D), lambda i, p: (0, 0)),
        ],
        out_specs=pl.BlockSpec((BQ, D), lambda i, p: (i, 0)),
        out_shape=jax.ShapeDtypeStruct((L, D), jnp.float32),
        scratch_shapes=[
            pltpu.VMEM((BQ, L), jnp.float32),
            pltpu.VMEM((NP, BQ, 2 * DH), jnp.float32),
        ],
    )(lw, fw, kp, cmr, qkv, qkv, qkv, x2, Wo.T, bo[None], Wg.T, bg[None])

    return y.reshape(1, L, D)
